# baseline (speedup 1.0000x reference)
import functools

import jax
import jax.numpy as jnp
from jax import lax
from jax.experimental import pallas as pl
from jax.experimental.pallas import tpu as pltpu
from jax.experimental.pallas import tpu_sc as plsc

NC = 2
NS = 16
NW = NC * NS
LANES = 16
ECHUNK = 128

_sc_mesh = plsc.VectorSubcoreMesh(core_axis_name="c", subcore_axis_name="s")


def _wid():
    return lax.axis_index("c") * NS + lax.axis_index("s")


def _deg_kernel(dst_hbm, w_hbm, zeros_hbm, degA, degB, dst_v, w_v, sem,
                acc, *, e_per_tile):
    cid = lax.axis_index("c")
    sid = lax.axis_index("s")
    wid = cid * NS + sid
    base = wid * e_per_tile
    nchunks = e_per_tile // ECHUNK

    @pl.when(sid == 0)
    def _():
        pltpu.sync_copy(zeros_hbm, acc)
    plsc.subcore_barrier()

    def chunk_body(c, _):
        off = base + c * ECHUNK
        pltpu.sync_copy(dst_hbm.at[pl.ds(off, ECHUNK)], dst_v)
        pltpu.sync_copy(w_hbm.at[pl.ds(off, ECHUNK)], w_v)
        pltpu.sync_copy(w_v, acc.at[dst_v], add=True)
        return 0

    lax.fori_loop(0, nchunks, chunk_body, 0)
    plsc.subcore_barrier()

    @pl.when((sid == 0) & (cid == 0))
    def _():
        pltpu.sync_copy(acc, degA)

    @pl.when((sid == 0) & (cid == 1))
    def _():
        pltpu.sync_copy(acc, degB)


def _norm_kernel(src_hbm, dst_hbm, w_hbm, dinv_hbm, norm_hbm,
                 src_v, dst_v, w_v, out_v, dinv_v, sem, *, e_per_tile):
    wid = _wid()
    base = wid * e_per_tile
    nchunks = e_per_tile // ECHUNK

    pltpu.sync_copy(dinv_hbm, dinv_v)

    def chunk_body(c, _):
        off = base + c * ECHUNK
        pltpu.sync_copy(src_hbm.at[pl.ds(off, ECHUNK)], src_v)
        pltpu.sync_copy(dst_hbm.at[pl.ds(off, ECHUNK)], dst_v)
        pltpu.sync_copy(w_hbm.at[pl.ds(off, ECHUNK)], w_v)
        for k in range(ECHUNK // LANES):
            sv = plsc.load_gather(dinv_v, [src_v[pl.ds(k * LANES, LANES)]])
            dv = plsc.load_gather(dinv_v, [dst_v[pl.ds(k * LANES, LANES)]])
            out_v[pl.ds(k * LANES, LANES)] = sv * w_v[pl.ds(k * LANES, LANES)] * dv
        pltpu.sync_copy(out_v, norm_hbm.at[pl.ds(off, ECHUNK)])
        return 0

    lax.fori_loop(0, nchunks, chunk_body, 0)


def _agg_kernel(xw_hbm, src_hbm, dst_hbm, w_hbm, dinv_hbm, zeros_hbm,
                aggA, aggB,
                src_v, dst_v, norm_v, rows_v, sdst_v, dinv_v,
                semA, semB, semS, acc, *, e_per_tile, d, chunk):
    cid = lax.axis_index("c")
    sid = lax.axis_index("s")
    wid = cid * NS + sid
    base = wid * e_per_tile
    nchunks = e_per_tile // chunk

    @pl.when(sid == 0)
    def _():
        pltpu.sync_copy(zeros_hbm, acc)
    pltpu.sync_copy(dinv_hbm, dinv_v)
    plsc.subcore_barrier()

    def startA(b, c):
        off = base + c * chunk
        pltpu.async_copy(src_hbm.at[pl.ds(off, chunk)], src_v[b], semA[b])
        pltpu.async_copy(dst_hbm.at[pl.ds(off, chunk)], dst_v[b], semA[b])
        pltpu.async_copy(w_hbm.at[pl.ds(off, chunk)], norm_v[b], semA[b])

    def waitA(b):
        pltpu.make_async_copy(src_hbm.at[pl.ds(0, chunk)], src_v[b],
                              semA[b]).wait()
        pltpu.make_async_copy(dst_hbm.at[pl.ds(0, chunk)], dst_v[b],
                              semA[b]).wait()
        pltpu.make_async_copy(w_hbm.at[pl.ds(0, chunk)], norm_v[b],
                              semA[b]).wait()

    def startB(b):
        pltpu.async_copy(xw_hbm.at[src_v[b]], rows_v[b], semB[b])

    def waitB(b):
        pltpu.make_async_copy(xw_hbm.at[src_v[b]], rows_v[b], semB[b]).wait()

    def scale(b):
        for k in range(chunk // LANES):
            sl = pl.ds(k * LANES, LANES)
            sv = plsc.load_gather(dinv_v, [src_v[b][sl]])
            dv = plsc.load_gather(dinv_v, [dst_v[b][sl]])
            norm_v[b][sl] = sv * norm_v[b][sl] * dv

        def scale_body(e, _):
            nb = plsc.load_gather(norm_v[b],
                                  [jnp.zeros((LANES,), jnp.int32) + e])
            for j in range(d // LANES):
                rows_v[b][e, pl.ds(j * LANES, LANES)] = (
                    rows_v[b][e, pl.ds(j * LANES, LANES)] * nb)
            return 0
        lax.fori_loop(0, chunk, scale_body, 0)

    def snap_dst(b):
        for j in range(chunk // LANES):
            sdst_v[b][pl.ds(j * LANES, LANES)] = dst_v[b][pl.ds(j * LANES,
                                                                LANES)]

    def startS(b):
        pltpu.async_copy(rows_v[b], acc.at[sdst_v[b]], semS[b], add=True)

    def waitS(b):
        pltpu.make_async_copy(rows_v[b], acc.at[sdst_v[b]], semS[b]).wait()

    startA(0, 0)
    startA(1, 1)

    def pair_body(g, _):
        for b in (0, 1):
            c = 2 * g + b
            waitA(b)

            @pl.when(g >= 1)
            def _():
                waitS(b)
            startB(b)

            def prev():
                waitB(1 - b)
                scale(1 - b)
                snap_dst(1 - b)
                startS(1 - b)
            if b == 1:
                prev()

                @pl.when(g < nchunks // 2 - 1)
                def _():
                    startA(0, c + 1)
            else:
                @pl.when(g >= 1)
                def _():
                    prev()
                    startA(1, c + 1)
        return 0

    lax.fori_loop(0, nchunks // 2, pair_body, 0)
    waitB(1)
    scale(1)
    snap_dst(1)
    startS(1)
    waitS(0)
    waitS(1)
    plsc.subcore_barrier()

    @pl.when((sid == 0) & (cid == 0))
    def _():
        pltpu.sync_copy(acc, aggA)

    @pl.when((sid == 0) & (cid == 1))
    def _():
        pltpu.sync_copy(acc, aggB)


def _sc_deg(dst, w, n):
    e_per_tile = dst.shape[0] // NW
    zeros = jnp.zeros((n,), jnp.float32)
    return pl.kernel(
        functools.partial(_deg_kernel, e_per_tile=e_per_tile),
        out_type=(jax.ShapeDtypeStruct((n,), jnp.float32),
                  jax.ShapeDtypeStruct((n,), jnp.float32)),
        mesh=_sc_mesh,
        compiler_params=pltpu.CompilerParams(needs_layout_passes=False),
        scratch_types=[
            pltpu.VMEM((ECHUNK,), jnp.int32),
            pltpu.VMEM((ECHUNK,), jnp.float32),
            pltpu.SemaphoreType.DMA,
            pltpu.VMEM_SHARED((n,), jnp.float32),
        ],
    )(dst, w, zeros)


def _sc_norm(src, dst, w, dinv):
    e = src.shape[0]
    e_per_tile = e // NW
    n = dinv.shape[0]
    return pl.kernel(
        functools.partial(_norm_kernel, e_per_tile=e_per_tile),
        out_type=jax.ShapeDtypeStruct((e,), jnp.float32),
        mesh=_sc_mesh,
        compiler_params=pltpu.CompilerParams(needs_layout_passes=False),
        scratch_types=[
            pltpu.VMEM((ECHUNK,), jnp.int32),
            pltpu.VMEM((ECHUNK,), jnp.int32),
            pltpu.VMEM((ECHUNK,), jnp.float32),
            pltpu.VMEM((ECHUNK,), jnp.float32),
            pltpu.VMEM((n,), jnp.float32),
            pltpu.SemaphoreType.DMA,
        ],
    )(src, dst, w, dinv)


def _sc_agg(xw, src, dst, w, dinv, chunk=128):
    n, d = xw.shape
    e_per_tile = src.shape[0] // NW
    zeros = jnp.zeros((n, d), jnp.float32)

    def two(t):
        return (t, t)

    return pl.kernel(
        functools.partial(_agg_kernel, e_per_tile=e_per_tile, d=d,
                          chunk=chunk),
        out_type=(jax.ShapeDtypeStruct((n, d), jnp.float32),
                  jax.ShapeDtypeStruct((n, d), jnp.float32)),
        mesh=_sc_mesh,
        compiler_params=pltpu.CompilerParams(needs_layout_passes=False),
        scratch_types=[
            two(pltpu.VMEM((chunk,), jnp.int32)),
            two(pltpu.VMEM((chunk,), jnp.int32)),
            two(pltpu.VMEM((chunk,), jnp.float32)),
            two(pltpu.VMEM((chunk, d), jnp.float32)),
            two(pltpu.VMEM((chunk,), jnp.int32)),
            pltpu.VMEM((n,), jnp.float32),
            two(pltpu.SemaphoreType.DMA),
            two(pltpu.SemaphoreType.DMA),
            two(pltpu.SemaphoreType.DMA),
            pltpu.VMEM_SHARED((n, d), jnp.float32),
        ],
    )(xw, src, dst, w, dinv, zeros)


def _gather_kernel(table_hbm, idx_hbm, out_hbm, idx_v, rows_v, tbl_s,
                   semA, semB, semS, *, rows_per_tile, d, chunk):
    sid = lax.axis_index("s")
    wid = lax.axis_index("c") * NS + sid
    base = wid * rows_per_tile
    nchunks = rows_per_tile // chunk

    @pl.when(sid == 0)
    def _():
        pltpu.sync_copy(table_hbm, tbl_s)
    plsc.subcore_barrier()

    def startA(b, c):
        pltpu.async_copy(idx_hbm.at[pl.ds(base + c * chunk, chunk)],
                         idx_v[b], semA[b])

    def waitA(b):
        pltpu.make_async_copy(idx_hbm.at[pl.ds(0, chunk)], idx_v[b],
                              semA[b]).wait()

    def startB(b):
        pltpu.async_copy(tbl_s.at[idx_v[b]], rows_v[b], semB[b])

    def waitB(b):
        pltpu.make_async_copy(tbl_s.at[idx_v[b]], rows_v[b], semB[b]).wait()

    def startS(b, c):
        pltpu.async_copy(rows_v[b], out_hbm.at[pl.ds(base + c * chunk,
                                                     chunk)], semS[b])

    def waitS(b):
        pltpu.make_async_copy(rows_v[b], out_hbm.at[pl.ds(0, chunk)],
                              semS[b]).wait()

    startA(0, 0)
    startA(1, 1)

    def pair_body(g, _):
        for b in (0, 1):
            c = 2 * g + b
            waitA(b)

            @pl.when(g >= 1)
            def _():
                waitS(b)
            startB(b)

            def prev():
                waitB(1 - b)
                startS(1 - b, c - 1)
            if b == 1:
                prev()

                @pl.when(g < nchunks // 2 - 1)
                def _():
                    startA(0, c + 1)
            else:
                @pl.when(g >= 1)
                def _():
                    prev()
                    startA(1, c + 1)
        return 0

    lax.fori_loop(0, nchunks // 2, pair_body, 0)
    waitB(1)
    startS(1, nchunks - 1)
    waitS(0)
    waitS(1)


def _sc_gather(table, idx, chunk=80):
    v, d = table.shape
    m = idx.shape[0]
    rows_per_tile = m // NW

    def two(t):
        return (t, t)

    return pl.kernel(
        functools.partial(_gather_kernel, rows_per_tile=rows_per_tile, d=d,
                          chunk=chunk),
        out_type=jax.ShapeDtypeStruct((m, d), jnp.float32),
        mesh=_sc_mesh,
        compiler_params=pltpu.CompilerParams(needs_layout_passes=False),
        scratch_types=[
            two(pltpu.VMEM((chunk,), jnp.int32)),
            two(pltpu.VMEM((chunk, d), jnp.float32)),
            pltpu.VMEM_SHARED((v, d), jnp.float32),
            two(pltpu.SemaphoreType.DMA),
            two(pltpu.SemaphoreType.DMA),
            two(pltpu.SemaphoreType.DMA),
        ],
    )(table, idx)


BN = 1000


def _dinv_kernel(degA_ref, degB_ref, dinv_ref, invdeg_ref):
    deg = degA_ref[...] + degB_ref[...] + 1.0
    dinv_ref[...] = lax.rsqrt(deg)
    invdeg_ref[...] = 1.0 / deg


def _tc_dinv(degA, degB):
    n = degA.shape[0]
    return pl.pallas_call(
        _dinv_kernel,
        grid=(n // BN,),
        in_specs=[pl.BlockSpec((BN, 1), lambda i: (i, 0)),
                  pl.BlockSpec((BN, 1), lambda i: (i, 0))],
        out_specs=[pl.BlockSpec((BN, 1), lambda i: (i, 0)),
                   pl.BlockSpec((BN, 1), lambda i: (i, 0))],
        out_shape=[jax.ShapeDtypeStruct((n, 1), jnp.float32),
                   jax.ShapeDtypeStruct((n, 1), jnp.float32)],
    )(degA, degB)


def _mm2_kernel(x_ref, wa_ref, ba_ref, wb_ref, out_ref):
    h = jnp.dot(x_ref[...], wa_ref[...],
                preferred_element_type=jnp.float32) + ba_ref[...]
    out_ref[...] = jnp.dot(h, wb_ref[...], preferred_element_type=jnp.float32)


def _tc_fc_chain(x, Wa, ba, Wb):
    n, k = x.shape
    d = Wa.shape[1]
    d2 = Wb.shape[1]
    return pl.pallas_call(
        _mm2_kernel,
        grid=(n // BN,),
        in_specs=[pl.BlockSpec((BN, k), lambda i: (i, 0)),
                  pl.BlockSpec((k, d), lambda i: (0, 0)),
                  pl.BlockSpec((1, d), lambda i: (0, 0)),
                  pl.BlockSpec((d, d2), lambda i: (0, 0))],
        out_specs=pl.BlockSpec((BN, d2), lambda i: (i, 0)),
        out_shape=jax.ShapeDtypeStruct((n, d2), jnp.float32),
    )(x, Wa, ba[None, :], Wb)


def _mm1_kernel(x_ref, w_ref, b_ref, out_ref):
    out_ref[...] = jnp.dot(x_ref[...], w_ref[...],
                           preferred_element_type=jnp.float32) + b_ref[...]


def _tc_fc(x, W, b):
    n, k = x.shape
    d = W.shape[1]
    return pl.pallas_call(
        _mm1_kernel,
        grid=(n // BN,),
        in_specs=[pl.BlockSpec((BN, k), lambda i: (i, 0)),
                  pl.BlockSpec((k, d), lambda i: (0, 0)),
                  pl.BlockSpec((1, d), lambda i: (0, 0))],
        out_specs=pl.BlockSpec((BN, d), lambda i: (i, 0)),
        out_shape=jax.ShapeDtypeStruct((n, d), jnp.float32),
    )(x, W, b[None, :])


def _post_kernel(aggA_ref, aggB_ref, xw_ref, invdeg_ref, b_ref, g_ref,
                 beta_ref, wn_ref, out_ref):
    y = (aggA_ref[...] + aggB_ref[...] + xw_ref[...] * invdeg_ref[...]
         + b_ref[...])
    r = jnp.maximum(y, 0.0)
    mu = jnp.mean(r, axis=1, keepdims=True)
    var = jnp.mean((r - mu) ** 2, axis=1, keepdims=True)
    xn = (r - mu) * lax.rsqrt(var + 1e-5) * g_ref[...] + beta_ref[...]
    out_ref[...] = jnp.dot(xn, wn_ref[...], preferred_element_type=jnp.float32)


def _tc_post(aggA, aggB, xw, invdeg, b, g, beta, Wnext):
    n, d = xw.shape
    return pl.pallas_call(
        _post_kernel,
        grid=(n // BN,),
        in_specs=[pl.BlockSpec((BN, d), lambda i: (i, 0)),
                  pl.BlockSpec((BN, d), lambda i: (i, 0)),
                  pl.BlockSpec((BN, d), lambda i: (i, 0)),
                  pl.BlockSpec((BN, 1), lambda i: (i, 0)),
                  pl.BlockSpec((1, d), lambda i: (0, 0)),
                  pl.BlockSpec((1, d), lambda i: (0, 0)),
                  pl.BlockSpec((1, d), lambda i: (0, 0)),
                  pl.BlockSpec((d, d), lambda i: (0, 0))],
        out_specs=pl.BlockSpec((BN, d), lambda i: (i, 0)),
        out_shape=jax.ShapeDtypeStruct((n, d), jnp.float32),
    )(aggA, aggB, xw, invdeg, b[None, :], g[None, :], beta[None, :], Wnext)


def _final_kernel(aggA_ref, aggB_ref, xw_ref, invdeg_ref, b_ref,
                  w1_ref, b1_ref, w2_ref, b2_ref, emb_ref, out_ref):
    y = (aggA_ref[...] + aggB_ref[...] + xw_ref[...] * invdeg_ref[...]
         + b_ref[...])
    r = jnp.maximum(y, 0.0)
    emb_ref[...] = r
    h = jnp.dot(r, w1_ref[...], preferred_element_type=jnp.float32) + b1_ref[...]
    z = jnp.dot(h, w2_ref[...], preferred_element_type=jnp.float32) + b2_ref[...]
    m = jnp.max(z, axis=1, keepdims=True)
    lse = jnp.log(jnp.sum(jnp.exp(z - m), axis=1, keepdims=True)) + m
    out_ref[...] = z - lse


def _tc_final(aggA, aggB, xw, invdeg, b, W1, b1, W2, b2):
    n, d = xw.shape
    out = W2.shape[1]
    return pl.pallas_call(
        _final_kernel,
        grid=(n // BN,),
        in_specs=[pl.BlockSpec((BN, d), lambda i: (i, 0)),
                  pl.BlockSpec((BN, d), lambda i: (i, 0)),
                  pl.BlockSpec((BN, d), lambda i: (i, 0)),
                  pl.BlockSpec((BN, 1), lambda i: (i, 0)),
                  pl.BlockSpec((1, d), lambda i: (0, 0)),
                  pl.BlockSpec((d, d), lambda i: (0, 0)),
                  pl.BlockSpec((1, d), lambda i: (0, 0)),
                  pl.BlockSpec((d, out), lambda i: (0, 0)),
                  pl.BlockSpec((1, out), lambda i: (0, 0))],
        out_specs=[pl.BlockSpec((BN, d), lambda i: (i, 0)),
                   pl.BlockSpec((BN, out), lambda i: (i, 0))],
        out_shape=[jax.ShapeDtypeStruct((n, d), jnp.float32),
                   jax.ShapeDtypeStruct((n, out), jnp.float32)],
    )(aggA, aggB, xw, invdeg, b[None, :], W1, b1[None, :], W2, b2[None, :])


def _gru_kernel(seq_ref, len_ref, wih_ref, whh_ref, bih_ref, bhh_ref,
                out_ref, h_s, o_s, *, t_total, h):
    t = pl.program_id(1)

    @pl.when(t == 0)
    def _():
        h_s[...] = jnp.zeros_like(h_s)

    x_t = seq_ref[...]
    hp = h_s[...]
    gx = jnp.dot(x_t, wih_ref[...], preferred_element_type=jnp.float32) \
        + bih_ref[...]
    gh = jnp.dot(hp, whh_ref[...], preferred_element_type=jnp.float32) \
        + bhh_ref[...]
    r = jax.nn.sigmoid(gx[:, :h] + gh[:, :h])
    z = jax.nn.sigmoid(gx[:, h:2 * h] + gh[:, h:2 * h])
    n = jnp.tanh(gx[:, 2 * h:] + r * gh[:, 2 * h:])
    hn = (1.0 - z) * n + z * hp
    h_s[...] = hn

    keep = len_ref[...] > lax.convert_element_type(t, jnp.float32)
    o = jnp.where(keep, hn, o_s[...])
    o_s[...] = o

    @pl.when(t == t_total - 1)
    def _():
        out_ref[...] = o


def _tc_gru(seq, lengths_f32, Wih, Whh, bih, bhh, bn):
    b = seq.shape[0]
    d = Wih.shape[1]
    t_total = seq.shape[1] // d
    h = Whh.shape[1]
    g3 = 3 * h
    return pl.pallas_call(
        functools.partial(_gru_kernel, t_total=t_total, h=h),
        grid=(b // bn, t_total),
        in_specs=[
            pl.BlockSpec((bn, d), lambda i, t: (i, t)),
            pl.BlockSpec((bn, 1), lambda i, t: (i, 0)),
            pl.BlockSpec((d, g3), lambda i, t: (0, 0)),
            pl.BlockSpec((h, g3), lambda i, t: (0, 0)),
            pl.BlockSpec((1, g3), lambda i, t: (0, 0)),
            pl.BlockSpec((1, g3), lambda i, t: (0, 0)),
        ],
        out_specs=pl.BlockSpec((bn, h), lambda i, t: (i, 0)),
        out_shape=jax.ShapeDtypeStruct((b, h), jnp.float32),
        scratch_shapes=[pltpu.VMEM((bn, h), jnp.float32),
                        pltpu.VMEM((bn, h), jnp.float32)],
    )(seq, lengths_f32, Wih.T, Whh.T, bih[None, :], bhh[None, :])


def kernel(idx_lp, idx_ns, x_lp_length, x_ns_length, edge_index, edge_weight,
           x_ref, x_def, x_pdt, lp_emb, ns_emb,
           lp_Wih, lp_Whh, lp_bih, lp_bhh, lp_fc_W, lp_fc_b,
           ns_Wih, ns_Whh, ns_bih, ns_bhh, all_fc_W, all_fc_b,
           conv_W0, conv_b0, conv_W1, conv_b1, conv_W2, conv_b2,
           ln_g0, ln_b0, ln_g1, ln_b1, mp_W1, mp_b1, mp_W2, mp_b2):
    N, P, _ = idx_lp.shape
    D = lp_emb.shape[1]
    E = edge_index.shape[1]
    epad = (-E) % (NW * ECHUNK * 2)
    src = jnp.concatenate([edge_index[0], jnp.zeros((epad,), jnp.int32)])
    dst = jnp.concatenate([edge_index[1], jnp.zeros((epad,), jnp.int32)])
    ew = jnp.concatenate([edge_weight, jnp.zeros((epad,), jnp.float32)])

    degA, degB = _sc_deg(dst, ew, N)
    dinv2d, invdeg = _tc_dinv(degA.reshape(N, 1), degB.reshape(N, 1))
    dinv = dinv2d.reshape(N)

    T_LP = idx_lp.shape[2]
    T_NS = idx_ns.shape[1]
    idx_pm = jnp.transpose(idx_lp, (1, 0, 2)).reshape(P * N * T_LP)
    len_pm = jnp.transpose(x_lp_length, (1, 0)).reshape(P * N, 1)
    seq_lp = _sc_gather(lp_emb, idx_pm, chunk=80).reshape(P * N, T_LP * D)
    h_lp = _tc_gru(seq_lp, len_pm.astype(jnp.float32),
                   lp_Wih, lp_Whh, lp_bih, lp_bhh, bn=2000)
    x_lp = h_lp.reshape(N, P * D)
    x_lp = _tc_fc(x_lp, lp_fc_W.T, lp_fc_b)
    seq_ns = _sc_gather(ns_emb, idx_ns.reshape(N * T_NS),
                        chunk=40).reshape(N, T_NS * D)
    x_ns = _tc_gru(seq_ns, x_ns_length.reshape(N, 1).astype(jnp.float32),
                   ns_Wih, ns_Whh, ns_bih, ns_bhh, bn=2000)
    xcat = jnp.concatenate([x_pdt, x_ref, x_def, x_lp, x_ns],
                           axis=0).reshape(N, -1)
    xw = _tc_fc_chain(xcat, all_fc_W.T, all_fc_b, conv_W0)

    aggA, aggB = _sc_agg(xw, src, dst, ew, dinv)
    xw = _tc_post(aggA, aggB, xw, invdeg, conv_b0, ln_g0, ln_b0, conv_W1)
    aggA, aggB = _sc_agg(xw, src, dst, ew, dinv)
    xw = _tc_post(aggA, aggB, xw, invdeg, conv_b1, ln_g1, ln_b1, conv_W2)

    aggA, aggB = _sc_agg(xw, src, dst, ew, dinv)
    emb, out = _tc_final(aggA, aggB, xw, invdeg, conv_b2,
                         mp_W1.T, mp_b1, mp_W2.T, mp_b2)
    return emb, out


# speedup vs baseline: 1.3124x; 1.3124x over previous
import functools

import jax
import jax.numpy as jnp
from jax import lax
from jax.experimental import pallas as pl
from jax.experimental.pallas import tpu as pltpu
from jax.experimental.pallas import tpu_sc as plsc

NC = 2
NS = 16
NW = NC * NS
LANES = 16
ECHUNK = 128

_sc_mesh = plsc.VectorSubcoreMesh(core_axis_name="c", subcore_axis_name="s")


def _wid():
    return lax.axis_index("c") * NS + lax.axis_index("s")


def _deg_kernel(dst_hbm, w_hbm, zeros_hbm, degA, degB, dst_v, w_v, sem,
                acc, *, e_per_tile):
    cid = lax.axis_index("c")
    sid = lax.axis_index("s")
    wid = cid * NS + sid
    base = wid * e_per_tile
    nchunks = e_per_tile // ECHUNK

    @pl.when(sid == 0)
    def _():
        pltpu.sync_copy(zeros_hbm, acc)
    plsc.subcore_barrier()

    def chunk_body(c, _):
        off = base + c * ECHUNK
        pltpu.sync_copy(dst_hbm.at[pl.ds(off, ECHUNK)], dst_v)
        pltpu.sync_copy(w_hbm.at[pl.ds(off, ECHUNK)], w_v)
        pltpu.sync_copy(w_v, acc.at[dst_v], add=True)
        return 0

    lax.fori_loop(0, nchunks, chunk_body, 0)
    plsc.subcore_barrier()

    @pl.when((sid == 0) & (cid == 0))
    def _():
        pltpu.sync_copy(acc, degA)

    @pl.when((sid == 0) & (cid == 1))
    def _():
        pltpu.sync_copy(acc, degB)


def _norm_kernel(src_hbm, dst_hbm, w_hbm, dinv_hbm, norm_hbm,
                 src_v, dst_v, w_v, out_v, dinv_v, sem, *, e_per_tile):
    wid = _wid()
    base = wid * e_per_tile
    nchunks = e_per_tile // ECHUNK

    pltpu.sync_copy(dinv_hbm, dinv_v)

    def chunk_body(c, _):
        off = base + c * ECHUNK
        pltpu.sync_copy(src_hbm.at[pl.ds(off, ECHUNK)], src_v)
        pltpu.sync_copy(dst_hbm.at[pl.ds(off, ECHUNK)], dst_v)
        pltpu.sync_copy(w_hbm.at[pl.ds(off, ECHUNK)], w_v)
        for k in range(ECHUNK // LANES):
            sv = plsc.load_gather(dinv_v, [src_v[pl.ds(k * LANES, LANES)]])
            dv = plsc.load_gather(dinv_v, [dst_v[pl.ds(k * LANES, LANES)]])
            out_v[pl.ds(k * LANES, LANES)] = sv * w_v[pl.ds(k * LANES, LANES)] * dv
        pltpu.sync_copy(out_v, norm_hbm.at[pl.ds(off, ECHUNK)])
        return 0

    lax.fori_loop(0, nchunks, chunk_body, 0)


def _agg_kernel(xw_hbm, src_hbm, dst_hbm, w_hbm, dinv_hbm, zeros_hbm,
                aggA, aggB,
                src_v, dst_v, norm_v, rows_v, sdst_v, dinv_v,
                semA, semB, semS, acc, *, e_per_tile, d, chunk):
    cid = lax.axis_index("c")
    sid = lax.axis_index("s")
    wid = cid * NS + sid
    base = wid * e_per_tile
    nchunks = e_per_tile // chunk

    n_nodes = acc.shape[0]
    rows_a = ((n_nodes // NS) // 8) * 8
    rows_last = n_nodes - (NS - 1) * rows_a

    @pl.when(sid < NS - 1)
    def _():
        pltpu.sync_copy(zeros_hbm.at[pl.ds(sid * rows_a, rows_a)],
                        acc.at[pl.ds(sid * rows_a, rows_a)])

    @pl.when(sid == NS - 1)
    def _():
        pltpu.sync_copy(zeros_hbm.at[pl.ds((NS - 1) * rows_a, rows_last)],
                        acc.at[pl.ds((NS - 1) * rows_a, rows_last)])
    pltpu.sync_copy(dinv_hbm, dinv_v)
    plsc.subcore_barrier()

    def startA(b, c):
        off = base + c * chunk
        pltpu.async_copy(src_hbm.at[pl.ds(off, chunk)], src_v[b], semA[b])
        pltpu.async_copy(dst_hbm.at[pl.ds(off, chunk)], dst_v[b], semA[b])
        pltpu.async_copy(w_hbm.at[pl.ds(off, chunk)], norm_v[b], semA[b])

    def waitA(b):
        pltpu.make_async_copy(src_hbm.at[pl.ds(0, chunk)], src_v[b],
                              semA[b]).wait()
        pltpu.make_async_copy(dst_hbm.at[pl.ds(0, chunk)], dst_v[b],
                              semA[b]).wait()
        pltpu.make_async_copy(w_hbm.at[pl.ds(0, chunk)], norm_v[b],
                              semA[b]).wait()

    def startB(b):
        pltpu.async_copy(xw_hbm.at[src_v[b]], rows_v[b], semB[b])

    def waitB(b):
        pltpu.make_async_copy(xw_hbm.at[src_v[b]], rows_v[b], semB[b]).wait()

    def scale(b):
        for k in range(chunk // LANES):
            sl = pl.ds(k * LANES, LANES)
            sv = plsc.load_gather(dinv_v, [src_v[b][sl]])
            dv = plsc.load_gather(dinv_v, [dst_v[b][sl]])
            norm_v[b][sl] = sv * norm_v[b][sl] * dv

        def scale_body(e, _):
            nb = plsc.load_gather(norm_v[b],
                                  [jnp.zeros((LANES,), jnp.int32) + e])
            for j in range(d // LANES):
                rows_v[b][e, pl.ds(j * LANES, LANES)] = (
                    rows_v[b][e, pl.ds(j * LANES, LANES)] * nb)
            return 0
        lax.fori_loop(0, chunk, scale_body, 0)

    def snap_dst(b):
        for j in range(chunk // LANES):
            sdst_v[b][pl.ds(j * LANES, LANES)] = dst_v[b][pl.ds(j * LANES,
                                                                LANES)]

    def startS(b):
        pltpu.async_copy(rows_v[b], acc.at[sdst_v[b]], semS[b], add=True)

    def waitS(b):
        pltpu.make_async_copy(rows_v[b], acc.at[sdst_v[b]], semS[b]).wait()

    startA(0, 0)
    startA(1, 1)

    def pair_body(g, _):
        for b in (0, 1):
            c = 2 * g + b
            waitA(b)

            @pl.when(g >= 1)
            def _():
                waitS(b)
            startB(b)

            def prev():
                waitB(1 - b)
                scale(1 - b)
                snap_dst(1 - b)
                startS(1 - b)
            if b == 1:
                prev()

                @pl.when(g < nchunks // 2 - 1)
                def _():
                    startA(0, c + 1)
            else:
                @pl.when(g >= 1)
                def _():
                    prev()
                    startA(1, c + 1)
        return 0

    lax.fori_loop(0, nchunks // 2, pair_body, 0)
    waitB(1)
    scale(1)
    snap_dst(1)
    startS(1)
    waitS(0)
    waitS(1)
    plsc.subcore_barrier()

    out = [aggA, aggB]
    for core in (0, 1):
        @pl.when((cid == core) & (sid < NS - 1))
        def _(core=core):
            pltpu.sync_copy(acc.at[pl.ds(sid * rows_a, rows_a)],
                            out[core].at[pl.ds(sid * rows_a, rows_a)])

        @pl.when((cid == core) & (sid == NS - 1))
        def _(core=core):
            pltpu.sync_copy(
                acc.at[pl.ds((NS - 1) * rows_a, rows_last)],
                out[core].at[pl.ds((NS - 1) * rows_a, rows_last)])


def _sc_deg(dst, w, n):
    e_per_tile = dst.shape[0] // NW
    zeros = jnp.zeros((n,), jnp.float32)
    return pl.kernel(
        functools.partial(_deg_kernel, e_per_tile=e_per_tile),
        out_type=(jax.ShapeDtypeStruct((n,), jnp.float32),
                  jax.ShapeDtypeStruct((n,), jnp.float32)),
        mesh=_sc_mesh,
        compiler_params=pltpu.CompilerParams(needs_layout_passes=False),
        scratch_types=[
            pltpu.VMEM((ECHUNK,), jnp.int32),
            pltpu.VMEM((ECHUNK,), jnp.float32),
            pltpu.SemaphoreType.DMA,
            pltpu.VMEM_SHARED((n,), jnp.float32),
        ],
    )(dst, w, zeros)


def _sc_norm(src, dst, w, dinv):
    e = src.shape[0]
    e_per_tile = e // NW
    n = dinv.shape[0]
    return pl.kernel(
        functools.partial(_norm_kernel, e_per_tile=e_per_tile),
        out_type=jax.ShapeDtypeStruct((e,), jnp.float32),
        mesh=_sc_mesh,
        compiler_params=pltpu.CompilerParams(needs_layout_passes=False),
        scratch_types=[
            pltpu.VMEM((ECHUNK,), jnp.int32),
            pltpu.VMEM((ECHUNK,), jnp.int32),
            pltpu.VMEM((ECHUNK,), jnp.float32),
            pltpu.VMEM((ECHUNK,), jnp.float32),
            pltpu.VMEM((n,), jnp.float32),
            pltpu.SemaphoreType.DMA,
        ],
    )(src, dst, w, dinv)


def _sc_agg(xw, src, dst, w, dinv, chunk=128):
    n, d = xw.shape
    e_per_tile = src.shape[0] // NW
    zeros = jnp.zeros((n, d), jnp.float32)

    def two(t):
        return (t, t)

    return pl.kernel(
        functools.partial(_agg_kernel, e_per_tile=e_per_tile, d=d,
                          chunk=chunk),
        out_type=(jax.ShapeDtypeStruct((n, d), jnp.float32),
                  jax.ShapeDtypeStruct((n, d), jnp.float32)),
        mesh=_sc_mesh,
        compiler_params=pltpu.CompilerParams(needs_layout_passes=False),
        scratch_types=[
            two(pltpu.VMEM((chunk,), jnp.int32)),
            two(pltpu.VMEM((chunk,), jnp.int32)),
            two(pltpu.VMEM((chunk,), jnp.float32)),
            two(pltpu.VMEM((chunk, d), jnp.float32)),
            two(pltpu.VMEM((chunk,), jnp.int32)),
            pltpu.VMEM((n,), jnp.float32),
            two(pltpu.SemaphoreType.DMA),
            two(pltpu.SemaphoreType.DMA),
            two(pltpu.SemaphoreType.DMA),
            pltpu.VMEM_SHARED((n, d), jnp.float32),
        ],
    )(xw, src, dst, w, dinv, zeros)


def _gather_kernel(table_hbm, idx_hbm, out_hbm, idx_v, rows_v, tbl_s,
                   semA, semB, semS, *, rows_per_tile, d, chunk):
    sid = lax.axis_index("s")
    wid = lax.axis_index("c") * NS + sid
    base = wid * rows_per_tile
    nchunks = rows_per_tile // chunk

    @pl.when(sid == 0)
    def _():
        pltpu.sync_copy(table_hbm, tbl_s)
    plsc.subcore_barrier()

    def startA(b, c):
        pltpu.async_copy(idx_hbm.at[pl.ds(base + c * chunk, chunk)],
                         idx_v[b], semA[b])

    def waitA(b):
        pltpu.make_async_copy(idx_hbm.at[pl.ds(0, chunk)], idx_v[b],
                              semA[b]).wait()

    def startB(b):
        pltpu.async_copy(tbl_s.at[idx_v[b]], rows_v[b], semB[b])

    def waitB(b):
        pltpu.make_async_copy(tbl_s.at[idx_v[b]], rows_v[b], semB[b]).wait()

    def startS(b, c):
        pltpu.async_copy(rows_v[b], out_hbm.at[pl.ds(base + c * chunk,
                                                     chunk)], semS[b])

    def waitS(b):
        pltpu.make_async_copy(rows_v[b], out_hbm.at[pl.ds(0, chunk)],
                              semS[b]).wait()

    startA(0, 0)
    startA(1, 1)

    def pair_body(g, _):
        for b in (0, 1):
            c = 2 * g + b
            waitA(b)

            @pl.when(g >= 1)
            def _():
                waitS(b)
            startB(b)

            def prev():
                waitB(1 - b)
                startS(1 - b, c - 1)
            if b == 1:
                prev()

                @pl.when(g < nchunks // 2 - 1)
                def _():
                    startA(0, c + 1)
            else:
                @pl.when(g >= 1)
                def _():
                    prev()
                    startA(1, c + 1)
        return 0

    lax.fori_loop(0, nchunks // 2, pair_body, 0)
    waitB(1)
    startS(1, nchunks - 1)
    waitS(0)
    waitS(1)


def _sc_gather(table, idx, chunk=80):
    v, d = table.shape
    m = idx.shape[0]
    rows_per_tile = m // NW

    def two(t):
        return (t, t)

    return pl.kernel(
        functools.partial(_gather_kernel, rows_per_tile=rows_per_tile, d=d,
                          chunk=chunk),
        out_type=jax.ShapeDtypeStruct((m, d), jnp.float32),
        mesh=_sc_mesh,
        compiler_params=pltpu.CompilerParams(needs_layout_passes=False),
        scratch_types=[
            two(pltpu.VMEM((chunk,), jnp.int32)),
            two(pltpu.VMEM((chunk, d), jnp.float32)),
            pltpu.VMEM_SHARED((v, d), jnp.float32),
            two(pltpu.SemaphoreType.DMA),
            two(pltpu.SemaphoreType.DMA),
            two(pltpu.SemaphoreType.DMA),
        ],
    )(table, idx)


BN = 1000


def _dinv_kernel(degA_ref, degB_ref, dinv_ref, invdeg_ref):
    deg = degA_ref[...] + degB_ref[...] + 1.0
    dinv_ref[...] = lax.rsqrt(deg)
    invdeg_ref[...] = 1.0 / deg


def _tc_dinv(degA, degB):
    n = degA.shape[0]
    return pl.pallas_call(
        _dinv_kernel,
        grid=(n // BN,),
        in_specs=[pl.BlockSpec((BN, 1), lambda i: (i, 0)),
                  pl.BlockSpec((BN, 1), lambda i: (i, 0))],
        out_specs=[pl.BlockSpec((BN, 1), lambda i: (i, 0)),
                   pl.BlockSpec((BN, 1), lambda i: (i, 0))],
        out_shape=[jax.ShapeDtypeStruct((n, 1), jnp.float32),
                   jax.ShapeDtypeStruct((n, 1), jnp.float32)],
    )(degA, degB)


def _mm2_kernel(x_ref, wa_ref, ba_ref, wb_ref, out_ref):
    h = jnp.dot(x_ref[...], wa_ref[...],
                preferred_element_type=jnp.float32) + ba_ref[...]
    out_ref[...] = jnp.dot(h, wb_ref[...], preferred_element_type=jnp.float32)


def _tc_fc_chain(x, Wa, ba, Wb):
    n, k = x.shape
    d = Wa.shape[1]
    d2 = Wb.shape[1]
    return pl.pallas_call(
        _mm2_kernel,
        grid=(n // BN,),
        in_specs=[pl.BlockSpec((BN, k), lambda i: (i, 0)),
                  pl.BlockSpec((k, d), lambda i: (0, 0)),
                  pl.BlockSpec((1, d), lambda i: (0, 0)),
                  pl.BlockSpec((d, d2), lambda i: (0, 0))],
        out_specs=pl.BlockSpec((BN, d2), lambda i: (i, 0)),
        out_shape=jax.ShapeDtypeStruct((n, d2), jnp.float32),
    )(x, Wa, ba[None, :], Wb)


def _mm1_kernel(x_ref, w_ref, b_ref, out_ref):
    out_ref[...] = jnp.dot(x_ref[...], w_ref[...],
                           preferred_element_type=jnp.float32) + b_ref[...]


def _tc_fc(x, W, b):
    n, k = x.shape
    d = W.shape[1]
    return pl.pallas_call(
        _mm1_kernel,
        grid=(n // BN,),
        in_specs=[pl.BlockSpec((BN, k), lambda i: (i, 0)),
                  pl.BlockSpec((k, d), lambda i: (0, 0)),
                  pl.BlockSpec((1, d), lambda i: (0, 0))],
        out_specs=pl.BlockSpec((BN, d), lambda i: (i, 0)),
        out_shape=jax.ShapeDtypeStruct((n, d), jnp.float32),
    )(x, W, b[None, :])


def _post_kernel(aggA_ref, aggB_ref, xw_ref, invdeg_ref, b_ref, g_ref,
                 beta_ref, wn_ref, out_ref):
    y = (aggA_ref[...] + aggB_ref[...] + xw_ref[...] * invdeg_ref[...]
         + b_ref[...])
    r = jnp.maximum(y, 0.0)
    mu = jnp.mean(r, axis=1, keepdims=True)
    var = jnp.mean((r - mu) ** 2, axis=1, keepdims=True)
    xn = (r - mu) * lax.rsqrt(var + 1e-5) * g_ref[...] + beta_ref[...]
    out_ref[...] = jnp.dot(xn, wn_ref[...], preferred_element_type=jnp.float32)


def _tc_post(aggA, aggB, xw, invdeg, b, g, beta, Wnext):
    n, d = xw.shape
    return pl.pallas_call(
        _post_kernel,
        grid=(n // BN,),
        in_specs=[pl.BlockSpec((BN, d), lambda i: (i, 0)),
                  pl.BlockSpec((BN, d), lambda i: (i, 0)),
                  pl.BlockSpec((BN, d), lambda i: (i, 0)),
                  pl.BlockSpec((BN, 1), lambda i: (i, 0)),
                  pl.BlockSpec((1, d), lambda i: (0, 0)),
                  pl.BlockSpec((1, d), lambda i: (0, 0)),
                  pl.BlockSpec((1, d), lambda i: (0, 0)),
                  pl.BlockSpec((d, d), lambda i: (0, 0))],
        out_specs=pl.BlockSpec((BN, d), lambda i: (i, 0)),
        out_shape=jax.ShapeDtypeStruct((n, d), jnp.float32),
    )(aggA, aggB, xw, invdeg, b[None, :], g[None, :], beta[None, :], Wnext)


def _final_kernel(aggA_ref, aggB_ref, xw_ref, invdeg_ref, b_ref,
                  w1_ref, b1_ref, w2_ref, b2_ref, emb_ref, out_ref):
    y = (aggA_ref[...] + aggB_ref[...] + xw_ref[...] * invdeg_ref[...]
         + b_ref[...])
    r = jnp.maximum(y, 0.0)
    emb_ref[...] = r
    h = jnp.dot(r, w1_ref[...], preferred_element_type=jnp.float32) + b1_ref[...]
    z = jnp.dot(h, w2_ref[...], preferred_element_type=jnp.float32) + b2_ref[...]
    m = jnp.max(z, axis=1, keepdims=True)
    lse = jnp.log(jnp.sum(jnp.exp(z - m), axis=1, keepdims=True)) + m
    out_ref[...] = z - lse


def _tc_final(aggA, aggB, xw, invdeg, b, W1, b1, W2, b2):
    n, d = xw.shape
    out = W2.shape[1]
    return pl.pallas_call(
        _final_kernel,
        grid=(n // BN,),
        in_specs=[pl.BlockSpec((BN, d), lambda i: (i, 0)),
                  pl.BlockSpec((BN, d), lambda i: (i, 0)),
                  pl.BlockSpec((BN, d), lambda i: (i, 0)),
                  pl.BlockSpec((BN, 1), lambda i: (i, 0)),
                  pl.BlockSpec((1, d), lambda i: (0, 0)),
                  pl.BlockSpec((d, d), lambda i: (0, 0)),
                  pl.BlockSpec((1, d), lambda i: (0, 0)),
                  pl.BlockSpec((d, out), lambda i: (0, 0)),
                  pl.BlockSpec((1, out), lambda i: (0, 0))],
        out_specs=[pl.BlockSpec((BN, d), lambda i: (i, 0)),
                   pl.BlockSpec((BN, out), lambda i: (i, 0))],
        out_shape=[jax.ShapeDtypeStruct((n, d), jnp.float32),
                   jax.ShapeDtypeStruct((n, out), jnp.float32)],
    )(aggA, aggB, xw, invdeg, b[None, :], W1, b1[None, :], W2, b2[None, :])


def _gru_kernel(seq_ref, len_ref, wih_ref, whh_ref, bih_ref, bhh_ref,
                out_ref, h_s, o_s, *, t_total, h):
    t = pl.program_id(1)

    @pl.when(t == 0)
    def _():
        h_s[...] = jnp.zeros_like(h_s)

    x_t = seq_ref[0]
    hp = h_s[...]
    gx = jnp.dot(x_t, wih_ref[...], preferred_element_type=jnp.float32) \
        + bih_ref[...]
    gh = jnp.dot(hp, whh_ref[...], preferred_element_type=jnp.float32) \
        + bhh_ref[...]
    r = jax.nn.sigmoid(gx[:, :h] + gh[:, :h])
    z = jax.nn.sigmoid(gx[:, h:2 * h] + gh[:, h:2 * h])
    n = jnp.tanh(gx[:, 2 * h:] + r * gh[:, 2 * h:])
    hn = (1.0 - z) * n + z * hp
    h_s[...] = hn

    keep = len_ref[...] > lax.convert_element_type(t, jnp.float32)
    o = jnp.where(keep, hn, o_s[...])
    o_s[...] = o

    @pl.when(t == t_total - 1)
    def _():
        out_ref[...] = o


def _tc_gru(seq, lengths_f32, Wih, Whh, bih, bhh, bn):
    t_total, b, d = seq.shape
    h = Whh.shape[1]
    g3 = 3 * h
    return pl.pallas_call(
        functools.partial(_gru_kernel, t_total=t_total, h=h),
        grid=(b // bn, t_total),
        in_specs=[
            pl.BlockSpec((1, bn, d), lambda i, t: (t, i, 0)),
            pl.BlockSpec((bn, 1), lambda i, t: (i, 0)),
            pl.BlockSpec((d, g3), lambda i, t: (0, 0)),
            pl.BlockSpec((h, g3), lambda i, t: (0, 0)),
            pl.BlockSpec((1, g3), lambda i, t: (0, 0)),
            pl.BlockSpec((1, g3), lambda i, t: (0, 0)),
        ],
        out_specs=pl.BlockSpec((bn, h), lambda i, t: (i, 0)),
        out_shape=jax.ShapeDtypeStruct((b, h), jnp.float32),
        scratch_shapes=[pltpu.VMEM((bn, h), jnp.float32),
                        pltpu.VMEM((bn, h), jnp.float32)],
    )(seq, lengths_f32, Wih.T, Whh.T, bih[None, :], bhh[None, :])


def kernel(idx_lp, idx_ns, x_lp_length, x_ns_length, edge_index, edge_weight,
           x_ref, x_def, x_pdt, lp_emb, ns_emb,
           lp_Wih, lp_Whh, lp_bih, lp_bhh, lp_fc_W, lp_fc_b,
           ns_Wih, ns_Whh, ns_bih, ns_bhh, all_fc_W, all_fc_b,
           conv_W0, conv_b0, conv_W1, conv_b1, conv_W2, conv_b2,
           ln_g0, ln_b0, ln_g1, ln_b1, mp_W1, mp_b1, mp_W2, mp_b2):
    N, P, _ = idx_lp.shape
    D = lp_emb.shape[1]
    E = edge_index.shape[1]
    epad = (-E) % (NW * ECHUNK * 2)
    src = jnp.concatenate([edge_index[0], jnp.zeros((epad,), jnp.int32)])
    dst = jnp.concatenate([edge_index[1], jnp.zeros((epad,), jnp.int32)])
    ew = jnp.concatenate([edge_weight, jnp.zeros((epad,), jnp.float32)])

    degA, degB = _sc_deg(dst, ew, N)
    dinv2d, invdeg = _tc_dinv(degA.reshape(N, 1), degB.reshape(N, 1))
    dinv = dinv2d.reshape(N)

    T_LP = idx_lp.shape[2]
    T_NS = idx_ns.shape[1]
    idx_tm = jnp.transpose(idx_lp, (2, 1, 0)).reshape(T_LP * P * N)
    len_pm = jnp.transpose(x_lp_length, (1, 0)).reshape(P * N, 1)
    seq_lp = _sc_gather(lp_emb, idx_tm, chunk=80).reshape(T_LP, P * N, D)
    h_lp = _tc_gru(seq_lp, len_pm.astype(jnp.float32),
                   lp_Wih, lp_Whh, lp_bih, lp_bhh, bn=2000)
    x_lp = h_lp.reshape(N, P * D)
    x_lp = _tc_fc(x_lp, lp_fc_W.T, lp_fc_b)
    seq_ns = _sc_gather(ns_emb, jnp.transpose(idx_ns, (1, 0)).reshape(
        T_NS * N), chunk=40).reshape(T_NS, N, D)
    x_ns = _tc_gru(seq_ns, x_ns_length.reshape(N, 1).astype(jnp.float32),
                   ns_Wih, ns_Whh, ns_bih, ns_bhh, bn=2000)
    xcat = jnp.concatenate([x_pdt, x_ref, x_def, x_lp, x_ns],
                           axis=0).reshape(N, -1)
    xw = _tc_fc_chain(xcat, all_fc_W.T, all_fc_b, conv_W0)

    aggA, aggB = _sc_agg(xw, src, dst, ew, dinv)
    xw = _tc_post(aggA, aggB, xw, invdeg, conv_b0, ln_g0, ln_b0, conv_W1)
    aggA, aggB = _sc_agg(xw, src, dst, ew, dinv)
    xw = _tc_post(aggA, aggB, xw, invdeg, conv_b1, ln_g1, ln_b1, conv_W2)

    aggA, aggB = _sc_agg(xw, src, dst, ew, dinv)
    emb, out = _tc_final(aggA, aggB, xw, invdeg, conv_b2,
                         mp_W1.T, mp_b1, mp_W2.T, mp_b2)
    return emb, out


# speedup vs baseline: 2.0152x; 1.5356x over previous
import functools

import jax
import jax.numpy as jnp
from jax import lax
from jax.experimental import pallas as pl
from jax.experimental.pallas import tpu as pltpu
from jax.experimental.pallas import tpu_sc as plsc

NC = 2
NS = 16
NW = NC * NS
LANES = 16
ECHUNK = 128

_sc_mesh = plsc.VectorSubcoreMesh(core_axis_name="c", subcore_axis_name="s")


def _wid():
    return lax.axis_index("c") * NS + lax.axis_index("s")


def _deg_kernel(dst_hbm, w_hbm, zeros_hbm, degA, degB, dst_v, w_v, sem,
                acc, *, e_per_tile):
    cid = lax.axis_index("c")
    sid = lax.axis_index("s")
    wid = cid * NS + sid
    base = wid * e_per_tile
    nchunks = e_per_tile // ECHUNK

    @pl.when(sid == 0)
    def _():
        pltpu.sync_copy(zeros_hbm, acc)
    plsc.subcore_barrier()

    def chunk_body(c, _):
        off = base + c * ECHUNK
        pltpu.sync_copy(dst_hbm.at[pl.ds(off, ECHUNK)], dst_v)
        pltpu.sync_copy(w_hbm.at[pl.ds(off, ECHUNK)], w_v)
        pltpu.sync_copy(w_v, acc.at[dst_v], add=True)
        return 0

    lax.fori_loop(0, nchunks, chunk_body, 0)
    plsc.subcore_barrier()

    @pl.when((sid == 0) & (cid == 0))
    def _():
        pltpu.sync_copy(acc, degA)

    @pl.when((sid == 0) & (cid == 1))
    def _():
        pltpu.sync_copy(acc, degB)


def _norm_kernel(src_hbm, dst_hbm, w_hbm, dinv_hbm, norm_hbm,
                 src_v, dst_v, w_v, out_v, dinv_v, sem, *, e_per_tile):
    wid = _wid()
    base = wid * e_per_tile
    nchunks = e_per_tile // ECHUNK

    pltpu.sync_copy(dinv_hbm, dinv_v)

    def chunk_body(c, _):
        off = base + c * ECHUNK
        pltpu.sync_copy(src_hbm.at[pl.ds(off, ECHUNK)], src_v)
        pltpu.sync_copy(dst_hbm.at[pl.ds(off, ECHUNK)], dst_v)
        pltpu.sync_copy(w_hbm.at[pl.ds(off, ECHUNK)], w_v)
        for k in range(ECHUNK // LANES):
            sv = plsc.load_gather(dinv_v, [src_v[pl.ds(k * LANES, LANES)]])
            dv = plsc.load_gather(dinv_v, [dst_v[pl.ds(k * LANES, LANES)]])
            out_v[pl.ds(k * LANES, LANES)] = sv * w_v[pl.ds(k * LANES, LANES)] * dv
        pltpu.sync_copy(out_v, norm_hbm.at[pl.ds(off, ECHUNK)])
        return 0

    lax.fori_loop(0, nchunks, chunk_body, 0)


def _agg_kernel(xw_hbm, src_hbm, dst_hbm, w_hbm, dinv_hbm, zeros_hbm,
                aggA, aggB,
                src_v, dst_v, norm_v, rows_v, sdst_v, dinv_v,
                semA, semB, semS, acc, *, e_per_tile, d, chunk):
    cid = lax.axis_index("c")
    sid = lax.axis_index("s")
    wid = cid * NS + sid
    base = wid * e_per_tile
    nchunks = e_per_tile // chunk

    n_nodes = acc.shape[0]
    rows_a = ((n_nodes // NS) // 8) * 8
    rows_last = n_nodes - (NS - 1) * rows_a

    @pl.when(sid < NS - 1)
    def _():
        pltpu.sync_copy(zeros_hbm.at[pl.ds(sid * rows_a, rows_a)],
                        acc.at[pl.ds(sid * rows_a, rows_a)])

    @pl.when(sid == NS - 1)
    def _():
        pltpu.sync_copy(zeros_hbm.at[pl.ds((NS - 1) * rows_a, rows_last)],
                        acc.at[pl.ds((NS - 1) * rows_a, rows_last)])
    pltpu.sync_copy(dinv_hbm, dinv_v)
    plsc.subcore_barrier()

    def startA(b, c):
        off = base + c * chunk
        pltpu.async_copy(src_hbm.at[pl.ds(off, chunk)], src_v[b], semA[b])
        pltpu.async_copy(dst_hbm.at[pl.ds(off, chunk)], dst_v[b], semA[b])
        pltpu.async_copy(w_hbm.at[pl.ds(off, chunk)], norm_v[b], semA[b])

    def waitA(b):
        pltpu.make_async_copy(src_hbm.at[pl.ds(0, chunk)], src_v[b],
                              semA[b]).wait()
        pltpu.make_async_copy(dst_hbm.at[pl.ds(0, chunk)], dst_v[b],
                              semA[b]).wait()
        pltpu.make_async_copy(w_hbm.at[pl.ds(0, chunk)], norm_v[b],
                              semA[b]).wait()

    def startB(b):
        pltpu.async_copy(xw_hbm.at[src_v[b]], rows_v[b], semB[b])

    def waitB(b):
        pltpu.make_async_copy(xw_hbm.at[src_v[b]], rows_v[b], semB[b]).wait()

    def scale(b):
        for k in range(chunk // LANES):
            sl = pl.ds(k * LANES, LANES)
            sv = plsc.load_gather(dinv_v, [src_v[b][sl]])
            dv = plsc.load_gather(dinv_v, [dst_v[b][sl]])
            norm_v[b][sl] = sv * norm_v[b][sl] * dv

        def scale_body(e, _):
            nb = plsc.load_gather(norm_v[b],
                                  [jnp.zeros((LANES,), jnp.int32) + e])
            for j in range(d // LANES):
                rows_v[b][e, pl.ds(j * LANES, LANES)] = (
                    rows_v[b][e, pl.ds(j * LANES, LANES)] * nb)
            return 0
        lax.fori_loop(0, chunk, scale_body, 0)

    def snap_dst(b):
        for j in range(chunk // LANES):
            sdst_v[b][pl.ds(j * LANES, LANES)] = dst_v[b][pl.ds(j * LANES,
                                                                LANES)]

    def startS(b):
        pltpu.async_copy(rows_v[b], acc.at[sdst_v[b]], semS[b], add=True)

    def waitS(b):
        pltpu.make_async_copy(rows_v[b], acc.at[sdst_v[b]], semS[b]).wait()

    startA(0, 0)
    startA(1, 1)

    def pair_body(g, _):
        for b in (0, 1):
            c = 2 * g + b
            waitA(b)

            @pl.when(g >= 1)
            def _():
                waitS(b)
            startB(b)

            def prev():
                waitB(1 - b)
                scale(1 - b)
                snap_dst(1 - b)
                startS(1 - b)
            if b == 1:
                prev()

                @pl.when(g < nchunks // 2 - 1)
                def _():
                    startA(0, c + 1)
            else:
                @pl.when(g >= 1)
                def _():
                    prev()
                    startA(1, c + 1)
        return 0

    lax.fori_loop(0, nchunks // 2, pair_body, 0)
    waitB(1)
    scale(1)
    snap_dst(1)
    startS(1)
    waitS(0)
    waitS(1)
    plsc.subcore_barrier()

    out = [aggA, aggB]
    for core in (0, 1):
        @pl.when((cid == core) & (sid < NS - 1))
        def _(core=core):
            pltpu.sync_copy(acc.at[pl.ds(sid * rows_a, rows_a)],
                            out[core].at[pl.ds(sid * rows_a, rows_a)])

        @pl.when((cid == core) & (sid == NS - 1))
        def _(core=core):
            pltpu.sync_copy(
                acc.at[pl.ds((NS - 1) * rows_a, rows_last)],
                out[core].at[pl.ds((NS - 1) * rows_a, rows_last)])


def _sc_deg(dst, w, n):
    e_per_tile = dst.shape[0] // NW
    zeros = jnp.zeros((n,), jnp.float32)
    return pl.kernel(
        functools.partial(_deg_kernel, e_per_tile=e_per_tile),
        out_type=(jax.ShapeDtypeStruct((n,), jnp.float32),
                  jax.ShapeDtypeStruct((n,), jnp.float32)),
        mesh=_sc_mesh,
        compiler_params=pltpu.CompilerParams(needs_layout_passes=False),
        scratch_types=[
            pltpu.VMEM((ECHUNK,), jnp.int32),
            pltpu.VMEM((ECHUNK,), jnp.float32),
            pltpu.SemaphoreType.DMA,
            pltpu.VMEM_SHARED((n,), jnp.float32),
        ],
    )(dst, w, zeros)


def _sc_norm(src, dst, w, dinv):
    e = src.shape[0]
    e_per_tile = e // NW
    n = dinv.shape[0]
    return pl.kernel(
        functools.partial(_norm_kernel, e_per_tile=e_per_tile),
        out_type=jax.ShapeDtypeStruct((e,), jnp.float32),
        mesh=_sc_mesh,
        compiler_params=pltpu.CompilerParams(needs_layout_passes=False),
        scratch_types=[
            pltpu.VMEM((ECHUNK,), jnp.int32),
            pltpu.VMEM((ECHUNK,), jnp.int32),
            pltpu.VMEM((ECHUNK,), jnp.float32),
            pltpu.VMEM((ECHUNK,), jnp.float32),
            pltpu.VMEM((n,), jnp.float32),
            pltpu.SemaphoreType.DMA,
        ],
    )(src, dst, w, dinv)


def _sc_agg(xw, src, dst, w, dinv, chunk=128):
    n, d = xw.shape
    e_per_tile = src.shape[0] // NW
    zeros = jnp.zeros((n, d), jnp.float32)

    def two(t):
        return (t, t)

    return pl.kernel(
        functools.partial(_agg_kernel, e_per_tile=e_per_tile, d=d,
                          chunk=chunk),
        out_type=(jax.ShapeDtypeStruct((n, d), jnp.float32),
                  jax.ShapeDtypeStruct((n, d), jnp.float32)),
        mesh=_sc_mesh,
        compiler_params=pltpu.CompilerParams(needs_layout_passes=False),
        scratch_types=[
            two(pltpu.VMEM((chunk,), jnp.int32)),
            two(pltpu.VMEM((chunk,), jnp.int32)),
            two(pltpu.VMEM((chunk,), jnp.float32)),
            two(pltpu.VMEM((chunk, d), jnp.float32)),
            two(pltpu.VMEM((chunk,), jnp.int32)),
            pltpu.VMEM((n,), jnp.float32),
            two(pltpu.SemaphoreType.DMA),
            two(pltpu.SemaphoreType.DMA),
            two(pltpu.SemaphoreType.DMA),
            pltpu.VMEM_SHARED((n, d), jnp.float32),
        ],
    )(xw, src, dst, w, dinv, zeros)


def _gather_kernel(table_hbm, idx_hbm, out_hbm, idx_v, rows_v, tbl_s,
                   semA, semB, semS, *, rows_per_tile, d, chunk):
    sid = lax.axis_index("s")
    wid = lax.axis_index("c") * NS + sid
    base = wid * rows_per_tile
    nchunks = rows_per_tile // chunk

    @pl.when(sid == 0)
    def _():
        pltpu.sync_copy(table_hbm, tbl_s)
    plsc.subcore_barrier()

    def startA(b, c):
        pltpu.async_copy(idx_hbm.at[pl.ds(base + c * chunk, chunk)],
                         idx_v[b], semA[b])

    def waitA(b):
        pltpu.make_async_copy(idx_hbm.at[pl.ds(0, chunk)], idx_v[b],
                              semA[b]).wait()

    def startB(b):
        pltpu.async_copy(tbl_s.at[idx_v[b]], rows_v[b], semB[b])

    def waitB(b):
        pltpu.make_async_copy(tbl_s.at[idx_v[b]], rows_v[b], semB[b]).wait()

    def startS(b, c):
        pltpu.async_copy(rows_v[b], out_hbm.at[pl.ds(base + c * chunk,
                                                     chunk)], semS[b])

    def waitS(b):
        pltpu.make_async_copy(rows_v[b], out_hbm.at[pl.ds(0, chunk)],
                              semS[b]).wait()

    startA(0, 0)
    startA(1, 1)

    def pair_body(g, _):
        for b in (0, 1):
            c = 2 * g + b
            waitA(b)

            @pl.when(g >= 1)
            def _():
                waitS(b)
            startB(b)

            def prev():
                waitB(1 - b)
                startS(1 - b, c - 1)
            if b == 1:
                prev()

                @pl.when(g < nchunks // 2 - 1)
                def _():
                    startA(0, c + 1)
            else:
                @pl.when(g >= 1)
                def _():
                    prev()
                    startA(1, c + 1)
        return 0

    lax.fori_loop(0, nchunks // 2, pair_body, 0)
    waitB(1)
    startS(1, nchunks - 1)
    waitS(0)
    waitS(1)


def _sc_gather(table, idx, chunk=80):
    v, d = table.shape
    m = idx.shape[0]
    rows_per_tile = m // NW

    def two(t):
        return (t, t)

    return pl.kernel(
        functools.partial(_gather_kernel, rows_per_tile=rows_per_tile, d=d,
                          chunk=chunk),
        out_type=jax.ShapeDtypeStruct((m, d), jnp.float32),
        mesh=_sc_mesh,
        compiler_params=pltpu.CompilerParams(needs_layout_passes=False),
        scratch_types=[
            two(pltpu.VMEM((chunk,), jnp.int32)),
            two(pltpu.VMEM((chunk, d), jnp.float32)),
            pltpu.VMEM_SHARED((v, d), jnp.float32),
            two(pltpu.SemaphoreType.DMA),
            two(pltpu.SemaphoreType.DMA),
            two(pltpu.SemaphoreType.DMA),
        ],
    )(table, idx)


BN = 1000


def _dinv_kernel(degA_ref, degB_ref, dinv_ref, invdeg_ref):
    deg = degA_ref[...] + degB_ref[...] + 1.0
    dinv_ref[...] = lax.rsqrt(deg)
    invdeg_ref[...] = 1.0 / deg


def _tc_dinv(degA, degB):
    n = degA.shape[0]
    return pl.pallas_call(
        _dinv_kernel,
        grid=(n // BN,),
        in_specs=[pl.BlockSpec((BN, 1), lambda i: (i, 0)),
                  pl.BlockSpec((BN, 1), lambda i: (i, 0))],
        out_specs=[pl.BlockSpec((BN, 1), lambda i: (i, 0)),
                   pl.BlockSpec((BN, 1), lambda i: (i, 0))],
        out_shape=[jax.ShapeDtypeStruct((n, 1), jnp.float32),
                   jax.ShapeDtypeStruct((n, 1), jnp.float32)],
    )(degA, degB)


def _mm2_kernel(x_ref, wa_ref, ba_ref, wb_ref, out_ref):
    h = jnp.dot(x_ref[...], wa_ref[...],
                preferred_element_type=jnp.float32) + ba_ref[...]
    out_ref[...] = jnp.dot(h, wb_ref[...], preferred_element_type=jnp.float32)


def _tc_fc_chain(x, Wa, ba, Wb):
    n, k = x.shape
    d = Wa.shape[1]
    d2 = Wb.shape[1]
    return pl.pallas_call(
        _mm2_kernel,
        grid=(n // BN,),
        in_specs=[pl.BlockSpec((BN, k), lambda i: (i, 0)),
                  pl.BlockSpec((k, d), lambda i: (0, 0)),
                  pl.BlockSpec((1, d), lambda i: (0, 0)),
                  pl.BlockSpec((d, d2), lambda i: (0, 0))],
        out_specs=pl.BlockSpec((BN, d2), lambda i: (i, 0)),
        out_shape=jax.ShapeDtypeStruct((n, d2), jnp.float32),
    )(x, Wa, ba[None, :], Wb)


def _mm1_kernel(x_ref, w_ref, b_ref, out_ref):
    out_ref[...] = jnp.dot(x_ref[...], w_ref[...],
                           preferred_element_type=jnp.float32) + b_ref[...]


def _tc_fc(x, W, b):
    n, k = x.shape
    d = W.shape[1]
    return pl.pallas_call(
        _mm1_kernel,
        grid=(n // BN,),
        in_specs=[pl.BlockSpec((BN, k), lambda i: (i, 0)),
                  pl.BlockSpec((k, d), lambda i: (0, 0)),
                  pl.BlockSpec((1, d), lambda i: (0, 0))],
        out_specs=pl.BlockSpec((BN, d), lambda i: (i, 0)),
        out_shape=jax.ShapeDtypeStruct((n, d), jnp.float32),
    )(x, W, b[None, :])


def _post_kernel(aggA_ref, aggB_ref, xw_ref, invdeg_ref, b_ref, g_ref,
                 beta_ref, wn_ref, out_ref):
    y = (aggA_ref[...] + aggB_ref[...] + xw_ref[...] * invdeg_ref[...]
         + b_ref[...])
    r = jnp.maximum(y, 0.0)
    mu = jnp.mean(r, axis=1, keepdims=True)
    var = jnp.mean((r - mu) ** 2, axis=1, keepdims=True)
    xn = (r - mu) * lax.rsqrt(var + 1e-5) * g_ref[...] + beta_ref[...]
    out_ref[...] = jnp.dot(xn, wn_ref[...], preferred_element_type=jnp.float32)


def _tc_post(aggA, aggB, xw, invdeg, b, g, beta, Wnext):
    n, d = xw.shape
    return pl.pallas_call(
        _post_kernel,
        grid=(n // BN,),
        in_specs=[pl.BlockSpec((BN, d), lambda i: (i, 0)),
                  pl.BlockSpec((BN, d), lambda i: (i, 0)),
                  pl.BlockSpec((BN, d), lambda i: (i, 0)),
                  pl.BlockSpec((BN, 1), lambda i: (i, 0)),
                  pl.BlockSpec((1, d), lambda i: (0, 0)),
                  pl.BlockSpec((1, d), lambda i: (0, 0)),
                  pl.BlockSpec((1, d), lambda i: (0, 0)),
                  pl.BlockSpec((d, d), lambda i: (0, 0))],
        out_specs=pl.BlockSpec((BN, d), lambda i: (i, 0)),
        out_shape=jax.ShapeDtypeStruct((n, d), jnp.float32),
    )(aggA, aggB, xw, invdeg, b[None, :], g[None, :], beta[None, :], Wnext)


def _final_kernel(aggA_ref, aggB_ref, xw_ref, invdeg_ref, b_ref,
                  w1_ref, b1_ref, w2_ref, b2_ref, emb_ref, out_ref):
    y = (aggA_ref[...] + aggB_ref[...] + xw_ref[...] * invdeg_ref[...]
         + b_ref[...])
    r = jnp.maximum(y, 0.0)
    emb_ref[...] = r
    h = jnp.dot(r, w1_ref[...], preferred_element_type=jnp.float32) + b1_ref[...]
    z = jnp.dot(h, w2_ref[...], preferred_element_type=jnp.float32) + b2_ref[...]
    m = jnp.max(z, axis=1, keepdims=True)
    lse = jnp.log(jnp.sum(jnp.exp(z - m), axis=1, keepdims=True)) + m
    out_ref[...] = z - lse


def _tc_final(aggA, aggB, xw, invdeg, b, W1, b1, W2, b2):
    n, d = xw.shape
    out = W2.shape[1]
    return pl.pallas_call(
        _final_kernel,
        grid=(n // BN,),
        in_specs=[pl.BlockSpec((BN, d), lambda i: (i, 0)),
                  pl.BlockSpec((BN, d), lambda i: (i, 0)),
                  pl.BlockSpec((BN, d), lambda i: (i, 0)),
                  pl.BlockSpec((BN, 1), lambda i: (i, 0)),
                  pl.BlockSpec((1, d), lambda i: (0, 0)),
                  pl.BlockSpec((d, d), lambda i: (0, 0)),
                  pl.BlockSpec((1, d), lambda i: (0, 0)),
                  pl.BlockSpec((d, out), lambda i: (0, 0)),
                  pl.BlockSpec((1, out), lambda i: (0, 0))],
        out_specs=[pl.BlockSpec((BN, d), lambda i: (i, 0)),
                   pl.BlockSpec((BN, out), lambda i: (i, 0))],
        out_shape=[jax.ShapeDtypeStruct((n, d), jnp.float32),
                   jax.ShapeDtypeStruct((n, out), jnp.float32)],
    )(aggA, aggB, xw, invdeg, b[None, :], W1, b1[None, :], W2, b2[None, :])


def _gru_kernel(seq_ref, len_ref, wih_ref, whh_ref, bih_ref, bhh_ref,
                out_ref, h_s, o_s, *, t_total, h):
    t = pl.program_id(1)

    @pl.when(t == 0)
    def _():
        h_s[...] = jnp.zeros_like(h_s)

    x_t = seq_ref[0]
    hp = h_s[...]
    gx = jnp.dot(x_t, wih_ref[...], preferred_element_type=jnp.float32) \
        + bih_ref[...]
    gh = jnp.dot(hp, whh_ref[...], preferred_element_type=jnp.float32) \
        + bhh_ref[...]
    r = jax.nn.sigmoid(gx[:, :h] + gh[:, :h])
    z = jax.nn.sigmoid(gx[:, h:2 * h] + gh[:, h:2 * h])
    n = jnp.tanh(gx[:, 2 * h:] + r * gh[:, 2 * h:])
    hn = (1.0 - z) * n + z * hp
    h_s[...] = hn

    keep = len_ref[...] > lax.convert_element_type(t, jnp.float32)
    o = jnp.where(keep, hn, o_s[...])
    o_s[...] = o

    @pl.when(t == t_total - 1)
    def _():
        out_ref[...] = o


def _tc_gru(seq, lengths_f32, Wih, Whh, bih, bhh, bn):
    t_total, b, d = seq.shape
    h = Whh.shape[1]
    g3 = 3 * h
    return pl.pallas_call(
        functools.partial(_gru_kernel, t_total=t_total, h=h),
        grid=(b // bn, t_total),
        in_specs=[
            pl.BlockSpec((1, bn, d), lambda i, t: (t, i, 0)),
            pl.BlockSpec((bn, 1), lambda i, t: (i, 0)),
            pl.BlockSpec((d, g3), lambda i, t: (0, 0)),
            pl.BlockSpec((h, g3), lambda i, t: (0, 0)),
            pl.BlockSpec((1, g3), lambda i, t: (0, 0)),
            pl.BlockSpec((1, g3), lambda i, t: (0, 0)),
        ],
        out_specs=pl.BlockSpec((bn, h), lambda i, t: (i, 0)),
        out_shape=jax.ShapeDtypeStruct((b, h), jnp.float32),
        scratch_shapes=[pltpu.VMEM((bn, h), jnp.float32),
                        pltpu.VMEM((bn, h), jnp.float32)],
    )(seq, lengths_f32, Wih.T, Whh.T, bih[None, :], bhh[None, :])


def kernel(idx_lp, idx_ns, x_lp_length, x_ns_length, edge_index, edge_weight,
           x_ref, x_def, x_pdt, lp_emb, ns_emb,
           lp_Wih, lp_Whh, lp_bih, lp_bhh, lp_fc_W, lp_fc_b,
           ns_Wih, ns_Whh, ns_bih, ns_bhh, all_fc_W, all_fc_b,
           conv_W0, conv_b0, conv_W1, conv_b1, conv_W2, conv_b2,
           ln_g0, ln_b0, ln_g1, ln_b1, mp_W1, mp_b1, mp_W2, mp_b2):
    N, P, _ = idx_lp.shape
    D = lp_emb.shape[1]
    E = edge_index.shape[1]
    epad = (-E) % (NW * ECHUNK * 2)
    spread = (jnp.arange(epad, dtype=jnp.int32) * 97) % N
    src = jnp.concatenate([edge_index[0], spread])
    dst = jnp.concatenate([edge_index[1], spread])
    ew = jnp.concatenate([edge_weight, jnp.zeros((epad,), jnp.float32)])

    degA, degB = _sc_deg(dst, ew, N)
    dinv2d, invdeg = _tc_dinv(degA.reshape(N, 1), degB.reshape(N, 1))
    dinv = dinv2d.reshape(N)

    T_LP = idx_lp.shape[2]
    T_NS = idx_ns.shape[1]
    idx_tm = jnp.transpose(idx_lp, (2, 1, 0)).reshape(T_LP * P * N)
    len_pm = jnp.transpose(x_lp_length, (1, 0)).reshape(P * N, 1)
    seq_lp = _sc_gather(lp_emb, idx_tm, chunk=80).reshape(T_LP, P * N, D)
    h_lp = _tc_gru(seq_lp, len_pm.astype(jnp.float32),
                   lp_Wih, lp_Whh, lp_bih, lp_bhh, bn=2000)
    x_lp = h_lp.reshape(N, P * D)
    x_lp = _tc_fc(x_lp, lp_fc_W.T, lp_fc_b)
    seq_ns = _sc_gather(ns_emb, jnp.transpose(idx_ns, (1, 0)).reshape(
        T_NS * N), chunk=40).reshape(T_NS, N, D)
    x_ns = _tc_gru(seq_ns, x_ns_length.reshape(N, 1).astype(jnp.float32),
                   ns_Wih, ns_Whh, ns_bih, ns_bhh, bn=2000)
    xcat = jnp.concatenate([x_pdt, x_ref, x_def, x_lp, x_ns],
                           axis=0).reshape(N, -1)
    xw = _tc_fc_chain(xcat, all_fc_W.T, all_fc_b, conv_W0)

    aggA, aggB = _sc_agg(xw, src, dst, ew, dinv)
    xw = _tc_post(aggA, aggB, xw, invdeg, conv_b0, ln_g0, ln_b0, conv_W1)
    aggA, aggB = _sc_agg(xw, src, dst, ew, dinv)
    xw = _tc_post(aggA, aggB, xw, invdeg, conv_b1, ln_g1, ln_b1, conv_W2)

    aggA, aggB = _sc_agg(xw, src, dst, ew, dinv)
    emb, out = _tc_final(aggA, aggB, xw, invdeg, conv_b2,
                         mp_W1.T, mp_b1, mp_W2.T, mp_b2)
    return emb, out
